# baseline (speedup 1.0000x reference)
import jax
import jax.numpy as jnp
from jax import lax
from jax.experimental import pallas as pl
from jax.experimental.pallas import tpu as pltpu
from jax.experimental.pallas import tpu_sc as plsc

N = 100000
D = 128
G = 64
NC = 2
NS = 16
NW = NC * NS

N_SC = 48000
N_TC = N - N_SC
ROWS_PER_W = N_SC // NW
CHUNK = 125
NCHUNK = ROWS_PER_W // CHUNK

BLK = 2000
NB = N_TC // BLK


def _sc_segment_sum(feat, batch3, z128, z16, ones_h):
    mesh = plsc.VectorSubcoreMesh(
        core_axis_name="c", subcore_axis_name="s", num_cores=NC,
        num_subcores=NS)

    @pl.kernel(
        out_type=[
            jax.ShapeDtypeStruct((NC, G, D), jnp.float32),
            jax.ShapeDtypeStruct((NC, G, 16), jnp.float32),
        ],
        mesh=mesh,
        compiler_params=pltpu.CompilerParams(use_tc_tiling_on_sc=False),
        scratch_types=[
            pltpu.VMEM((NCHUNK, CHUNK), jnp.int32),
            pltpu.VMEM((CHUNK, D), jnp.float32),
            pltpu.VMEM((CHUNK, D), jnp.float32),
            pltpu.VMEM((CHUNK, 16), jnp.float32),
            pltpu.VMEM((G, D), jnp.float32),
            pltpu.VMEM((G, 16), jnp.float32),
            pltpu.VMEM_SHARED((G, D), jnp.float32),
            pltpu.VMEM_SHARED((G, 16), jnp.float32),
            pltpu.SemaphoreType.DMA,
            pltpu.SemaphoreType.DMA,
            pltpu.SemaphoreType.DMA,
        ],
    )
    def k(feat_h, batch_h, z128_h, z16_h, ones_hbm, sums_out, cnts_out,
          idx_v, buf0, buf1, ones_v, tmp, tmp16, acc, cnt,
          ld_sem0, ld_sem1, cnt_sem):
        c = lax.axis_index("c")
        s = lax.axis_index("s")
        wid = c * NS + s
        row0 = wid * ROWS_PER_W
        bufs = [buf0, buf1]
        sems = [ld_sem0, ld_sem1]

        loads = [None] * NCHUNK
        loads[0] = pltpu.async_copy(
            feat_h.at[pl.ds(row0, CHUNK)], buf0, ld_sem0)
        pltpu.sync_copy(batch_h.at[wid], idx_v)
        pltpu.sync_copy(ones_hbm, ones_v)

        @pl.when(s == 0)
        def _():
            pltpu.sync_copy(z128_h, tmp)
            pltpu.sync_copy(tmp, acc)
            pltpu.sync_copy(z16_h, tmp16)
            pltpu.sync_copy(tmp16, cnt)

        plsc.subcore_barrier()

        cnt_scatters = []
        for i in range(NCHUNK):
            if i + 1 < NCHUNK:
                loads[i + 1] = pltpu.async_copy(
                    feat_h.at[pl.ds(row0 + (i + 1) * CHUNK, CHUNK)],
                    bufs[(i + 1) % 2], sems[(i + 1) % 2])
            loads[i].wait()
            cnt_scatters.append(pltpu.async_copy(
                ones_v, cnt.at[idx_v.at[i]], cnt_sem, add=True))
            pltpu.sync_copy(bufs[i % 2], acc.at[idx_v.at[i]], add=True)
        for d in cnt_scatters:
            d.wait()
        plsc.subcore_barrier()

        @pl.when(s == 0)
        def _():
            pltpu.sync_copy(acc, tmp)
            pltpu.sync_copy(tmp, sums_out.at[c])
            pltpu.sync_copy(cnt, tmp16)
            pltpu.sync_copy(tmp16, cnts_out.at[c])

    return k(feat, batch3, z128, z16, ones_h)


def _tc_partial_body(feat_ref, batch_ref, sums_ref, cnts_ref):
    @pl.when(pl.program_id(0) == 0)
    def _():
        sums_ref[...] = jnp.zeros_like(sums_ref)
        cnts_ref[...] = jnp.zeros_like(cnts_ref)

    b = batch_ref[pl.program_id(0)][None, :]
    oh = (lax.broadcasted_iota(jnp.int32, (G, BLK), 0) == b).astype(
        jnp.float32)
    sums_ref[...] += jnp.dot(oh, feat_ref[...],
                             precision=lax.Precision.HIGHEST,
                             preferred_element_type=jnp.float32)
    cnts_ref[...] += jnp.broadcast_to(
        jnp.sum(oh, axis=1, keepdims=True), (G, D))


def _tc_partial(feat_tc, batch_tc):
    return pl.pallas_call(
        _tc_partial_body,
        grid=(NB,),
        in_specs=[
            pl.BlockSpec((BLK, D), lambda i: (i, 0)),
            pl.BlockSpec((NB, BLK), lambda i: (0, 0)),
        ],
        out_specs=[
            pl.BlockSpec((G, D), lambda i: (0, 0)),
            pl.BlockSpec((G, D), lambda i: (0, 0)),
        ],
        out_shape=[
            jax.ShapeDtypeStruct((G, D), jnp.float32),
            jax.ShapeDtypeStruct((G, D), jnp.float32),
        ],
    )(feat_tc, batch_tc)


def _mlp_body(sc_sums_ref, sc_cnts_ref, tc_sums_ref, tc_cnts_ref,
              w1_ref, b1_ref, w2_ref, b2_ref, out_ref):
    sums = sc_sums_ref[0] + sc_sums_ref[1] + tc_sums_ref[...]
    cnt = (sc_cnts_ref[0, :, 0] + sc_cnts_ref[1, :, 0]
           + tc_cnts_ref[:, 0])
    mean = sums / jnp.maximum(cnt, 1.0)[:, None]
    h = jnp.dot(mean, w1_ref[...], precision=lax.Precision.HIGHEST,
                preferred_element_type=jnp.float32) + b1_ref[0]
    alpha = 1.6732632423543772848170429916717
    scale = 1.0507009873554804934193349852946
    h = scale * jnp.where(h > 0, h, alpha * (jnp.exp(h) - 1.0))
    out_ref[...] = jnp.dot(h, w2_ref[...], precision=lax.Precision.HIGHEST,
                           preferred_element_type=jnp.float32) + b2_ref[0]


def kernel(node_invariant_features, batch, W1, b1, W2, b2):
    feat = node_invariant_features.astype(jnp.float32)
    batch = batch.astype(jnp.int32)
    batch3 = batch[:N_SC].reshape(NW, NCHUNK, CHUNK)
    z128 = jnp.zeros((G, D), jnp.float32)
    z16 = jnp.zeros((G, 16), jnp.float32)
    ones_h = jnp.ones((CHUNK, 16), jnp.float32)

    sc_sums, sc_cnts = _sc_segment_sum(feat[:N_SC], batch3, z128, z16, ones_h)
    tc_sums, tc_cnts = _tc_partial(feat[N_SC:], batch[N_SC:].reshape(NB, BLK))

    H = W1.shape[1]
    O = W2.shape[1]
    out = pl.pallas_call(
        _mlp_body,
        out_shape=jax.ShapeDtypeStruct((G, O), jnp.float32),
    )(sc_sums, sc_cnts, tc_sums, tc_cnts,
      W1, b1.reshape(1, H), W2, b2.reshape(1, O))
    return out


# speedup vs baseline: 1.6586x; 1.6586x over previous
import jax
import jax.numpy as jnp
from jax import lax
from jax.experimental import pallas as pl
from jax.experimental.pallas import tpu as pltpu
from jax.experimental.pallas import tpu_sc as plsc

N = 100000
D = 128
G = 64
NC = 2
NS = 16
NW = NC * NS

N_SC = 48000
N_TC = N - N_SC
ROWS_PER_W = N_SC // NW
CHUNK = 125
NCHUNK = ROWS_PER_W // CHUNK

BLK = 2000
NB = N_TC // BLK
TC_BLK0 = N_SC // BLK


def _sc_segment_sum(feat, batch3, z128, z16, ones_h):
    mesh = plsc.VectorSubcoreMesh(
        core_axis_name="c", subcore_axis_name="s", num_cores=NC,
        num_subcores=NS)

    @pl.kernel(
        out_type=[
            jax.ShapeDtypeStruct((NC, G, D), jnp.float32),
            jax.ShapeDtypeStruct((NC, G, 16), jnp.float32),
        ],
        mesh=mesh,
        compiler_params=pltpu.CompilerParams(use_tc_tiling_on_sc=False),
        scratch_types=[
            pltpu.VMEM((NCHUNK, CHUNK), jnp.int32),
            pltpu.VMEM((CHUNK, D), jnp.float32),
            pltpu.VMEM((CHUNK, D), jnp.float32),
            pltpu.VMEM((CHUNK, 16), jnp.float32),
            pltpu.VMEM((G, D), jnp.float32),
            pltpu.VMEM((G, 16), jnp.float32),
            pltpu.VMEM_SHARED((G, D), jnp.float32),
            pltpu.VMEM_SHARED((G, 16), jnp.float32),
            pltpu.SemaphoreType.DMA,
            pltpu.SemaphoreType.DMA,
            pltpu.SemaphoreType.DMA,
        ],
    )
    def k(feat_h, batch_h, z128_h, z16_h, ones_hbm, sums_out, cnts_out,
          idx_v, buf0, buf1, ones_v, tmp, tmp16, acc, cnt,
          ld_sem0, ld_sem1, cnt_sem):
        c = lax.axis_index("c")
        s = lax.axis_index("s")
        wid = c * NS + s
        row0 = wid * ROWS_PER_W
        bufs = [buf0, buf1]
        sems = [ld_sem0, ld_sem1]

        loads = [None] * NCHUNK
        loads[0] = pltpu.async_copy(
            feat_h.at[pl.ds(row0, CHUNK)], buf0, ld_sem0)
        pltpu.sync_copy(batch_h.at[wid], idx_v)
        pltpu.sync_copy(ones_hbm, ones_v)

        @pl.when(s == 0)
        def _():
            pltpu.sync_copy(z128_h, tmp)
            pltpu.sync_copy(tmp, acc)
            pltpu.sync_copy(z16_h, tmp16)
            pltpu.sync_copy(tmp16, cnt)

        plsc.subcore_barrier()

        cnt_scatters = []
        for i in range(NCHUNK):
            if i + 1 < NCHUNK:
                loads[i + 1] = pltpu.async_copy(
                    feat_h.at[pl.ds(row0 + (i + 1) * CHUNK, CHUNK)],
                    bufs[(i + 1) % 2], sems[(i + 1) % 2])
            loads[i].wait()
            cnt_scatters.append(pltpu.async_copy(
                ones_v, cnt.at[idx_v.at[i]], cnt_sem, add=True))
            pltpu.sync_copy(bufs[i % 2], acc.at[idx_v.at[i]], add=True)
        for d in cnt_scatters:
            d.wait()
        plsc.subcore_barrier()

        @pl.when(s == 0)
        def _():
            pltpu.sync_copy(acc, tmp)
            pltpu.sync_copy(tmp, sums_out.at[c])
            pltpu.sync_copy(cnt, tmp16)
            pltpu.sync_copy(tmp16, cnts_out.at[c])

    return k(feat, batch3, z128, z16, ones_h)


def _tc_partial_body(feat_ref, batch_ref, sums_ref, cnts_ref):
    @pl.when(pl.program_id(0) == 0)
    def _():
        sums_ref[...] = jnp.zeros_like(sums_ref)
        cnts_ref[...] = jnp.zeros_like(cnts_ref)

    b = batch_ref[pl.program_id(0)][None, :]
    oh = (lax.broadcasted_iota(jnp.int32, (G, BLK), 0) == b).astype(
        jnp.float32)
    sums_ref[...] += jnp.dot(oh, feat_ref[...],
                             precision=lax.Precision.HIGHEST,
                             preferred_element_type=jnp.float32)
    cnts_ref[...] += jnp.broadcast_to(
        jnp.sum(oh, axis=1, keepdims=True), (G, D))


def _tc_partial(feat_tc, batch_tc):
    return pl.pallas_call(
        _tc_partial_body,
        grid=(NB,),
        in_specs=[
            pl.BlockSpec((BLK, D), lambda i: (TC_BLK0 + i, 0)),
            pl.BlockSpec((NB, BLK), lambda i: (0, 0)),
        ],
        out_specs=[
            pl.BlockSpec((G, D), lambda i: (0, 0)),
            pl.BlockSpec((G, D), lambda i: (0, 0)),
        ],
        out_shape=[
            jax.ShapeDtypeStruct((G, D), jnp.float32),
            jax.ShapeDtypeStruct((G, D), jnp.float32),
        ],
    )(feat_tc, batch_tc)


def _mlp_body(sc_sums_ref, sc_cnts_ref, tc_sums_ref, tc_cnts_ref,
              w1_ref, b1_ref, w2_ref, b2_ref, out_ref):
    sums = sc_sums_ref[0] + sc_sums_ref[1] + tc_sums_ref[...]
    cnt = (sc_cnts_ref[0, :, 0] + sc_cnts_ref[1, :, 0]
           + tc_cnts_ref[:, 0])
    mean = sums / jnp.maximum(cnt, 1.0)[:, None]
    h = jnp.dot(mean, w1_ref[...], precision=lax.Precision.HIGHEST,
                preferred_element_type=jnp.float32) + b1_ref[0]
    alpha = 1.6732632423543772848170429916717
    scale = 1.0507009873554804934193349852946
    h = scale * jnp.where(h > 0, h, alpha * (jnp.exp(h) - 1.0))
    out_ref[...] = jnp.dot(h, w2_ref[...], precision=lax.Precision.HIGHEST,
                           preferred_element_type=jnp.float32) + b2_ref[0]


def kernel(node_invariant_features, batch, W1, b1, W2, b2):
    feat = node_invariant_features.astype(jnp.float32)
    batch = batch.astype(jnp.int32)
    batch3 = batch[:N_SC].reshape(NW, NCHUNK, CHUNK)
    z128 = jnp.zeros((G, D), jnp.float32)
    z16 = jnp.zeros((G, 16), jnp.float32)
    ones_h = jnp.ones((CHUNK, 16), jnp.float32)

    sc_sums, sc_cnts = _sc_segment_sum(feat, batch3, z128, z16, ones_h)
    tc_sums, tc_cnts = _tc_partial(feat, batch[N_SC:].reshape(NB, BLK))

    H = W1.shape[1]
    O = W2.shape[1]
    out = pl.pallas_call(
        _mlp_body,
        out_shape=jax.ShapeDtypeStruct((G, O), jnp.float32),
    )(sc_sums, sc_cnts, tc_sums, tc_cnts,
      W1, b1.reshape(1, H), W2, b2.reshape(1, O))
    return out


# speedup vs baseline: 1.7174x; 1.0354x over previous
import jax
import jax.numpy as jnp
from jax import lax
from jax.experimental import pallas as pl
from jax.experimental.pallas import tpu as pltpu
from jax.experimental.pallas import tpu_sc as plsc

N = 100000
D = 128
G = 64
NC = 2
NS = 16
NW = NC * NS

N_SC = 52000
N_TC = N - N_SC
ROWS_PER_W = N_SC // NW
CHUNK = 125
NCHUNK = ROWS_PER_W // CHUNK

BLK = 2000
NB = N_TC // BLK
TC_BLK0 = N_SC // BLK


def _sc_segment_sum(feat, batch3, z128, z16, ones_h):
    mesh = plsc.VectorSubcoreMesh(
        core_axis_name="c", subcore_axis_name="s", num_cores=NC,
        num_subcores=NS)

    @pl.kernel(
        out_type=[
            jax.ShapeDtypeStruct((NC, G, D), jnp.float32),
            jax.ShapeDtypeStruct((NC, G, 16), jnp.float32),
        ],
        mesh=mesh,
        compiler_params=pltpu.CompilerParams(use_tc_tiling_on_sc=False),
        scratch_types=[
            pltpu.VMEM((NCHUNK, CHUNK), jnp.int32),
            pltpu.VMEM((CHUNK, D), jnp.float32),
            pltpu.VMEM((CHUNK, D), jnp.float32),
            pltpu.VMEM((CHUNK, 16), jnp.float32),
            pltpu.VMEM((G, D), jnp.float32),
            pltpu.VMEM((G, 16), jnp.float32),
            pltpu.VMEM_SHARED((G, D), jnp.float32),
            pltpu.VMEM_SHARED((G, 16), jnp.float32),
            pltpu.SemaphoreType.DMA,
            pltpu.SemaphoreType.DMA,
            pltpu.SemaphoreType.DMA,
        ],
    )
    def k(feat_h, batch_h, z128_h, z16_h, ones_hbm, sums_out, cnts_out,
          idx_v, buf0, buf1, ones_v, tmp, tmp16, acc, cnt,
          ld_sem0, ld_sem1, cnt_sem):
        c = lax.axis_index("c")
        s = lax.axis_index("s")
        wid = c * NS + s
        row0 = wid * ROWS_PER_W
        bufs = [buf0, buf1]
        sems = [ld_sem0, ld_sem1]

        loads = [None] * NCHUNK
        loads[0] = pltpu.async_copy(
            feat_h.at[pl.ds(row0, CHUNK)], buf0, ld_sem0)
        pltpu.sync_copy(batch_h.at[wid], idx_v)
        pltpu.sync_copy(ones_hbm, ones_v)

        @pl.when(s == 0)
        def _():
            pltpu.sync_copy(z128_h, tmp)
            pltpu.sync_copy(tmp, acc)
            pltpu.sync_copy(z16_h, tmp16)
            pltpu.sync_copy(tmp16, cnt)

        plsc.subcore_barrier()

        cnt_scatters = []
        for i in range(NCHUNK):
            if i + 1 < NCHUNK:
                loads[i + 1] = pltpu.async_copy(
                    feat_h.at[pl.ds(row0 + (i + 1) * CHUNK, CHUNK)],
                    bufs[(i + 1) % 2], sems[(i + 1) % 2])
            loads[i].wait()
            cnt_scatters.append(pltpu.async_copy(
                ones_v, cnt.at[idx_v.at[i]], cnt_sem, add=True))
            pltpu.sync_copy(bufs[i % 2], acc.at[idx_v.at[i]], add=True)
        for d in cnt_scatters:
            d.wait()
        plsc.subcore_barrier()

        @pl.when(s == 0)
        def _():
            pltpu.sync_copy(acc, tmp)
            pltpu.sync_copy(tmp, sums_out.at[c])
            pltpu.sync_copy(cnt, tmp16)
            pltpu.sync_copy(tmp16, cnts_out.at[c])

    return k(feat, batch3, z128, z16, ones_h)


def _tc_partial_body(feat_ref, batch_ref, sums_ref, cnts_ref):
    @pl.when(pl.program_id(0) == 0)
    def _():
        sums_ref[...] = jnp.zeros_like(sums_ref)
        cnts_ref[...] = jnp.zeros_like(cnts_ref)

    b = batch_ref[pl.program_id(0)][None, :]
    oh = (lax.broadcasted_iota(jnp.int32, (G, BLK), 0) == b).astype(
        jnp.float32)
    sums_ref[...] += jnp.dot(oh, feat_ref[...],
                             precision=lax.Precision.HIGHEST,
                             preferred_element_type=jnp.float32)
    cnts_ref[...] += jnp.broadcast_to(
        jnp.sum(oh, axis=1, keepdims=True), (G, D))


def _tc_partial(feat_tc, batch_tc):
    return pl.pallas_call(
        _tc_partial_body,
        grid=(NB,),
        in_specs=[
            pl.BlockSpec((BLK, D), lambda i: (TC_BLK0 + i, 0)),
            pl.BlockSpec((NB, BLK), lambda i: (0, 0)),
        ],
        out_specs=[
            pl.BlockSpec((G, D), lambda i: (0, 0)),
            pl.BlockSpec((G, D), lambda i: (0, 0)),
        ],
        out_shape=[
            jax.ShapeDtypeStruct((G, D), jnp.float32),
            jax.ShapeDtypeStruct((G, D), jnp.float32),
        ],
    )(feat_tc, batch_tc)


def _mlp_body(sc_sums_ref, sc_cnts_ref, tc_sums_ref, tc_cnts_ref,
              w1_ref, b1_ref, w2_ref, b2_ref, out_ref):
    sums = sc_sums_ref[0] + sc_sums_ref[1] + tc_sums_ref[...]
    cnt = (sc_cnts_ref[0, :, 0] + sc_cnts_ref[1, :, 0]
           + tc_cnts_ref[:, 0])
    mean = sums / jnp.maximum(cnt, 1.0)[:, None]
    h = jnp.dot(mean, w1_ref[...], precision=lax.Precision.HIGHEST,
                preferred_element_type=jnp.float32) + b1_ref[0]
    alpha = 1.6732632423543772848170429916717
    scale = 1.0507009873554804934193349852946
    h = scale * jnp.where(h > 0, h, alpha * (jnp.exp(h) - 1.0))
    out_ref[...] = jnp.dot(h, w2_ref[...], precision=lax.Precision.HIGHEST,
                           preferred_element_type=jnp.float32) + b2_ref[0]


def kernel(node_invariant_features, batch, W1, b1, W2, b2):
    feat = node_invariant_features.astype(jnp.float32)
    batch = batch.astype(jnp.int32)
    batch3 = batch[:N_SC].reshape(NW, NCHUNK, CHUNK)
    z128 = jnp.zeros((G, D), jnp.float32)
    z16 = jnp.zeros((G, 16), jnp.float32)
    ones_h = jnp.ones((CHUNK, 16), jnp.float32)

    sc_sums, sc_cnts = _sc_segment_sum(feat, batch3, z128, z16, ones_h)
    tc_sums, tc_cnts = _tc_partial(feat, batch[N_SC:].reshape(NB, BLK))

    H = W1.shape[1]
    O = W2.shape[1]
    out = pl.pallas_call(
        _mlp_body,
        out_shape=jax.ShapeDtypeStruct((G, O), jnp.float32),
    )(sc_sums, sc_cnts, tc_sums, tc_cnts,
      W1, b1.reshape(1, H), W2, b2.reshape(1, O))
    return out


# speedup vs baseline: 1.8840x; 1.0970x over previous
import jax
import jax.numpy as jnp
from jax import lax
from jax.experimental import pallas as pl
from jax.experimental.pallas import tpu as pltpu
from jax.experimental.pallas import tpu_sc as plsc

N = 100000
D = 128
G = 64
NC = 2
NS = 16
NW = NC * NS

N_SC = 52000
N_TC = N - N_SC
ROWS_PER_W = N_SC // NW
CHUNK = 125
NCHUNK = ROWS_PER_W // CHUNK

BLK = 4000
NB = N_TC // BLK
TC_BLK0 = N_SC // BLK


def _sc_segment_sum(feat, batch3, z128, z16, ones_h):
    mesh = plsc.VectorSubcoreMesh(
        core_axis_name="c", subcore_axis_name="s", num_cores=NC,
        num_subcores=NS)

    @pl.kernel(
        out_type=[
            jax.ShapeDtypeStruct((NC, G, D), jnp.float32),
            jax.ShapeDtypeStruct((NC, G, 16), jnp.float32),
        ],
        mesh=mesh,
        compiler_params=pltpu.CompilerParams(use_tc_tiling_on_sc=False),
        scratch_types=[
            pltpu.VMEM((NCHUNK, CHUNK), jnp.int32),
            pltpu.VMEM((CHUNK, D), jnp.float32),
            pltpu.VMEM((CHUNK, D), jnp.float32),
            pltpu.VMEM((CHUNK, 16), jnp.float32),
            pltpu.VMEM((G, D), jnp.float32),
            pltpu.VMEM((G, 16), jnp.float32),
            pltpu.VMEM_SHARED((G, D), jnp.float32),
            pltpu.VMEM_SHARED((G, 16), jnp.float32),
            pltpu.SemaphoreType.DMA,
            pltpu.SemaphoreType.DMA,
            pltpu.SemaphoreType.DMA,
        ],
    )
    def k(feat_h, batch_h, z128_h, z16_h, ones_hbm, sums_out, cnts_out,
          idx_v, buf0, buf1, ones_v, tmp, tmp16, acc, cnt,
          ld_sem0, ld_sem1, cnt_sem):
        c = lax.axis_index("c")
        s = lax.axis_index("s")
        wid = c * NS + s
        row0 = wid * ROWS_PER_W
        bufs = [buf0, buf1]
        sems = [ld_sem0, ld_sem1]

        loads = [None] * NCHUNK
        loads[0] = pltpu.async_copy(
            feat_h.at[pl.ds(row0, CHUNK)], buf0, ld_sem0)
        pltpu.sync_copy(batch_h.at[wid], idx_v)
        pltpu.sync_copy(ones_hbm, ones_v)

        @pl.when(s == 0)
        def _():
            pltpu.sync_copy(z128_h, tmp)
            pltpu.sync_copy(tmp, acc)
            pltpu.sync_copy(z16_h, tmp16)
            pltpu.sync_copy(tmp16, cnt)

        plsc.subcore_barrier()

        cnt_scatters = []
        for i in range(NCHUNK):
            if i + 1 < NCHUNK:
                loads[i + 1] = pltpu.async_copy(
                    feat_h.at[pl.ds(row0 + (i + 1) * CHUNK, CHUNK)],
                    bufs[(i + 1) % 2], sems[(i + 1) % 2])
            loads[i].wait()
            cnt_scatters.append(pltpu.async_copy(
                ones_v, cnt.at[idx_v.at[i]], cnt_sem, add=True))
            pltpu.sync_copy(bufs[i % 2], acc.at[idx_v.at[i]], add=True)
        for d in cnt_scatters:
            d.wait()
        plsc.subcore_barrier()

        @pl.when(s == 0)
        def _():
            pltpu.sync_copy(acc, tmp)
            pltpu.sync_copy(tmp, sums_out.at[c])
            pltpu.sync_copy(cnt, tmp16)
            pltpu.sync_copy(tmp16, cnts_out.at[c])

    return k(feat, batch3, z128, z16, ones_h)


def _tc_partial_body(feat_ref, batch_ref, sums_ref, cnts_ref):
    @pl.when(pl.program_id(0) == 0)
    def _():
        sums_ref[...] = jnp.zeros_like(sums_ref)
        cnts_ref[...] = jnp.zeros_like(cnts_ref)

    b = batch_ref[pl.program_id(0)][None, :]
    oh = (lax.broadcasted_iota(jnp.int32, (G, BLK), 0) == b).astype(
        jnp.float32)
    sums_ref[...] += jnp.dot(oh, feat_ref[...],
                             precision=lax.Precision.HIGHEST,
                             preferred_element_type=jnp.float32)
    cnts_ref[...] += jnp.broadcast_to(
        jnp.sum(oh, axis=1, keepdims=True), (G, D))


def _tc_partial(feat_tc, batch_tc):
    return pl.pallas_call(
        _tc_partial_body,
        grid=(NB,),
        in_specs=[
            pl.BlockSpec((BLK, D), lambda i: (TC_BLK0 + i, 0)),
            pl.BlockSpec((NB, BLK), lambda i: (0, 0)),
        ],
        out_specs=[
            pl.BlockSpec((G, D), lambda i: (0, 0)),
            pl.BlockSpec((G, D), lambda i: (0, 0)),
        ],
        out_shape=[
            jax.ShapeDtypeStruct((G, D), jnp.float32),
            jax.ShapeDtypeStruct((G, D), jnp.float32),
        ],
    )(feat_tc, batch_tc)


def _mlp_body(sc_sums_ref, sc_cnts_ref, tc_sums_ref, tc_cnts_ref,
              w1_ref, b1_ref, w2_ref, b2_ref, out_ref):
    sums = sc_sums_ref[0] + sc_sums_ref[1] + tc_sums_ref[...]
    cnt = (sc_cnts_ref[0, :, 0] + sc_cnts_ref[1, :, 0]
           + tc_cnts_ref[:, 0])
    mean = sums / jnp.maximum(cnt, 1.0)[:, None]
    h = jnp.dot(mean, w1_ref[...], precision=lax.Precision.HIGHEST,
                preferred_element_type=jnp.float32) + b1_ref[0]
    alpha = 1.6732632423543772848170429916717
    scale = 1.0507009873554804934193349852946
    h = scale * jnp.where(h > 0, h, alpha * (jnp.exp(h) - 1.0))
    out_ref[...] = jnp.dot(h, w2_ref[...], precision=lax.Precision.HIGHEST,
                           preferred_element_type=jnp.float32) + b2_ref[0]


def kernel(node_invariant_features, batch, W1, b1, W2, b2):
    feat = node_invariant_features.astype(jnp.float32)
    batch = batch.astype(jnp.int32)
    batch3 = batch[:N_SC].reshape(NW, NCHUNK, CHUNK)
    z128 = jnp.zeros((G, D), jnp.float32)
    z16 = jnp.zeros((G, 16), jnp.float32)
    ones_h = jnp.ones((CHUNK, 16), jnp.float32)

    sc_sums, sc_cnts = _sc_segment_sum(feat, batch3, z128, z16, ones_h)
    tc_sums, tc_cnts = _tc_partial(feat, batch[N_SC:].reshape(NB, BLK))

    H = W1.shape[1]
    O = W2.shape[1]
    out = pl.pallas_call(
        _mlp_body,
        out_shape=jax.ShapeDtypeStruct((G, O), jnp.float32),
    )(sc_sums, sc_cnts, tc_sums, tc_cnts,
      W1, b1.reshape(1, H), W2, b2.reshape(1, O))
    return out
